# Initial kernel scaffold; baseline (speedup 1.0000x reference)
import jax
import jax.numpy as jnp
from jax.experimental import pallas as pl


def kernel(x, ptr, W_ih0, W_hh0, b_ih0, b_hh0, W_ih1, W_hh1, b_ih1, b_hh1):
    raise NotImplementedError("write your pallas kernel here")


# speedup vs baseline: 3.9190x; 3.9190x over previous
import functools
import jax
import jax.numpy as jnp
from jax import lax
from jax.experimental import pallas as pl
from jax.experimental.pallas import tpu as pltpu
from jax.experimental.pallas import tpu_sc as plsc

D = 128
B = 256
N = 100000
NW = 32
RPW = 3136
RPW_LAST = N - (NW - 1) * RPW
CH = 32
NEG = -3.0e38


def _sc_readout(x, ptr_pad, qh):
    mesh = plsc.VectorSubcoreMesh(core_axis_name="c", subcore_axis_name="s")

    @functools.partial(
        pl.kernel,
        out_type=(
            jax.ShapeDtypeStruct((NW, B), jnp.float32),
            jax.ShapeDtypeStruct((NW, B), jnp.float32),
            jax.ShapeDtypeStruct((NW, B, D), jnp.float32),
        ),
        mesh=mesh,
        scratch_types=[
            pltpu.VMEM((RPW,), jnp.int32),
            pltpu.VMEM((RPW,), jnp.float32),
            pltpu.VMEM((CH, D), jnp.float32),
            pltpu.VMEM((B, D), jnp.float32),
            pltpu.VMEM((B,), jnp.float32),
            pltpu.VMEM((B,), jnp.float32),
            pltpu.SMEM((B,), jnp.float32),
            pltpu.SMEM((B,), jnp.float32),
        ],
        compiler_params=pltpu.CompilerParams(needs_layout_passes=False),
    )
    def k(x_hbm, ptr_hbm, qh_hbm, m_hbm, s_hbm, r_hbm,
          pv, ev, xa, qr, mv, sv, msm, ssm):
        wid = lax.axis_index("s") * 2 + lax.axis_index("c")
        base = wid * RPW
        nch = lax.select(wid == NW - 1, RPW_LAST // CH, RPW // CH)
        lanes = lax.iota(jnp.int32, 16)

        pltpu.sync_copy(ptr_hbm.at[pl.ds(base, RPW)], pv)
        pltpu.sync_copy(qh_hbm, qr)

        def init_sm(i, _):
            msm[i] = jnp.float32(NEG)
            ssm[i] = jnp.float32(0.0)
            return 0
        lax.fori_loop(0, B, init_sm, 0)

        def pass_a(ch, _):
            pltpu.sync_copy(x_hbm.at[pl.ds(base + ch * CH, CH)], xa)
            for g2 in range(CH // 16):
                off = ch * CH + g2 * 16
                segv = pv[pl.ds(off, 16)]
                evec = jnp.zeros((16,), jnp.float32)
                for i in range(16):
                    seg = segv[i]
                    row = g2 * 16 + i
                    prods = []
                    for c in range(8):
                        xv = xa[row, pl.ds(c * 16, 16)]
                        qv = qr[seg, pl.ds(c * 16, 16)]
                        prods.append(xv * qv)
                    s0 = (prods[0] + prods[1]) + (prods[2] + prods[3])
                    s1 = (prods[4] + prods[5]) + (prods[6] + prods[7])
                    tot = jnp.sum(s0 + s1)
                    msm[seg] = jnp.maximum(msm[seg], tot)
                    evec = jnp.where(lanes == i, tot, evec)
                ev[pl.ds(off, 16)] = evec
            return 0
        lax.fori_loop(0, nch, pass_a, 0)

        def pub_m(blk, _):
            vec = jnp.zeros((16,), jnp.float32)
            for j in range(16):
                vec = jnp.where(lanes == j, msm[blk * 16 + j], vec)
            mv[pl.ds(blk * 16, 16)] = vec
            return 0
        lax.fori_loop(0, B // 16, pub_m, 0)

        def pass_a2(g, _):
            off = g * 16
            segv = pv[pl.ds(off, 16)]
            evec = ev[pl.ds(off, 16)]
            mg = plsc.load_gather(mv, [segv])
            ex = jnp.exp(evec - mg)
            ev[pl.ds(off, 16)] = ex
            for i in range(16):
                seg = segv[i]
                ssm[seg] = ssm[seg] + ex[i]
            return 0
        lax.fori_loop(0, nch * (CH // 16), pass_a2, 0)

        def pub_s(blk, _):
            vec = jnp.zeros((16,), jnp.float32)
            for j in range(16):
                vec = jnp.where(lanes == j, ssm[blk * 16 + j], vec)
            sv[pl.ds(blk * 16, 16)] = vec
            return 0
        lax.fori_loop(0, B // 16, pub_s, 0)
        pltpu.sync_copy(mv, m_hbm.at[wid])
        pltpu.sync_copy(sv, s_hbm.at[wid])

        zero16 = jnp.zeros((16,), jnp.float32)

        def zero_r(row, _):
            for c in range(8):
                qr[row, pl.ds(c * 16, 16)] = zero16
            return 0
        lax.fori_loop(0, B, zero_r, 0)

        def pass_b(ch, _):
            pltpu.sync_copy(x_hbm.at[pl.ds(base + ch * CH, CH)], xa)
            for g2 in range(CH // 16):
                off = ch * CH + g2 * 16
                segv = pv[pl.ds(off, 16)]
                exv = ev[pl.ds(off, 16)]
                for i in range(16):
                    seg = segv[i]
                    w = exv[i]
                    row = g2 * 16 + i
                    for c in range(8):
                        xv = xa[row, pl.ds(c * 16, 16)]
                        qr[seg, pl.ds(c * 16, 16)] = (
                            qr[seg, pl.ds(c * 16, 16)] + w * xv)
            return 0
        lax.fori_loop(0, nch, pass_b, 0)
        pltpu.sync_copy(qr, r_hbm.at[wid])

    return k(x, ptr_pad, qh)


def _tc_step(m_all, s_all, r_all, qh_prev, h0, c0, h1, c1,
             W_ih0, W_hh0, b_ih0, b_hh0, W_ih1, W_hh1, b_ih1, b_hh1):
    def body(m_ref, s_ref, r_ref, qh_ref, h0_ref, c0_ref, h1_ref, c1_ref,
             wi0_ref, wh0_ref, bi0_ref, bh0_ref, wi1_ref, wh1_ref, bi1_ref,
             bh1_ref, q_ref, qh_o, h0_o, c0_o, h1_o, c1_o):
        m_all = m_ref[...]
        M = jnp.max(m_all, axis=0, keepdims=True)
        scale = jnp.exp(m_all - M)
        s_tot = jnp.sum(scale * s_ref[...], axis=0, keepdims=True)
        safe = s_tot > 0
        inv = jnp.where(safe, 1.0 / jnp.where(safe, s_tot, 1.0), 0.0)
        scale2 = scale * inv
        scale3 = lax.broadcast_in_dim(scale2, (NW, B, D), (0, 1))
        r = jnp.sum(scale3 * r_ref[...], axis=0)
        q = jnp.concatenate([qh_ref[...], r], axis=-1)
        q_ref[...] = q

        dn = (((1,), (1,)), ((), ()))
        g = (lax.dot_general(q, wi0_ref[...], dn,
                             preferred_element_type=jnp.float32)
             + lax.dot_general(h0_ref[...], wh0_ref[...], dn,
                               preferred_element_type=jnp.float32)
             + lax.broadcast_in_dim(bi0_ref[...] + bh0_ref[...],
                                    (B, 4 * D), (1,)))
        i0 = jax.nn.sigmoid(g[:, 0:D])
        f0 = jax.nn.sigmoid(g[:, D:2 * D])
        g0 = jnp.tanh(g[:, 2 * D:3 * D])
        o0 = jax.nn.sigmoid(g[:, 3 * D:4 * D])
        c0n = f0 * c0_ref[...] + i0 * g0
        h0n = o0 * jnp.tanh(c0n)
        g = (lax.dot_general(h0n, wi1_ref[...], dn,
                             preferred_element_type=jnp.float32)
             + lax.dot_general(h1_ref[...], wh1_ref[...], dn,
                               preferred_element_type=jnp.float32)
             + lax.broadcast_in_dim(bi1_ref[...] + bh1_ref[...],
                                    (B, 4 * D), (1,)))
        i1 = jax.nn.sigmoid(g[:, 0:D])
        f1 = jax.nn.sigmoid(g[:, D:2 * D])
        g1 = jnp.tanh(g[:, 2 * D:3 * D])
        o1 = jax.nn.sigmoid(g[:, 3 * D:4 * D])
        c1n = f1 * c1_ref[...] + i1 * g1
        h1n = o1 * jnp.tanh(c1n)
        qh_o[...] = h1n
        h0_o[...] = h0n
        c0_o[...] = c0n
        h1_o[...] = h1n
        c1_o[...] = c1n

    shapes = (
        jax.ShapeDtypeStruct((B, 2 * D), jnp.float32),
        jax.ShapeDtypeStruct((B, D), jnp.float32),
        jax.ShapeDtypeStruct((B, D), jnp.float32),
        jax.ShapeDtypeStruct((B, D), jnp.float32),
        jax.ShapeDtypeStruct((B, D), jnp.float32),
        jax.ShapeDtypeStruct((B, D), jnp.float32),
    )
    return pl.pallas_call(body, out_shape=shapes)(
        m_all, s_all, r_all, qh_prev, h0, c0, h1, c1,
        W_ih0, W_hh0, b_ih0, b_hh0, W_ih1, W_hh1, b_ih1, b_hh1)


def kernel(x, ptr, W_ih0, W_hh0, b_ih0, b_hh0, W_ih1, W_hh1, b_ih1, b_hh1):
    ptr32 = ptr.astype(jnp.int32)
    ptr_pad = jnp.concatenate(
        [ptr32, jnp.zeros((NW * RPW - N,), jnp.int32)])

    zM = jnp.full((NW, B), NEG, jnp.float32)
    zS = jnp.zeros((NW, B), jnp.float32)
    zR = jnp.zeros((NW, B, D), jnp.float32)
    qh = jnp.zeros((B, D), jnp.float32)
    h0 = c0 = h1 = c1 = jnp.zeros((B, D), jnp.float32)

    m_all, s_all, r_all = zM, zS, zR
    q = None
    for it in range(3):
        q, qh, h0, c0, h1, c1 = _tc_step(
            m_all, s_all, r_all, qh, h0, c0, h1, c1,
            W_ih0, W_hh0, b_ih0, b_hh0, W_ih1, W_hh1, b_ih1, b_hh1)
        m_all, s_all, r_all = _sc_readout(x, ptr_pad, qh)
    q, _, _, _, _, _ = _tc_step(
        m_all, s_all, r_all, qh, h0, c0, h1, c1,
        W_ih0, W_hh0, b_ih0, b_hh0, W_ih1, W_hh1, b_ih1, b_hh1)
    return q


# speedup vs baseline: 5.8674x; 1.4972x over previous
import functools
import jax
import jax.numpy as jnp
from jax import lax
from jax.experimental import pallas as pl
from jax.experimental.pallas import tpu as pltpu
from jax.experimental.pallas import tpu_sc as plsc

D = 128
B = 256
N = 100000
NW = 32
RPW = 3136
RPW_LAST = N - (NW - 1) * RPW
CH = 32
NEG = -3.0e38


def _sc_readout(x, ptr_pad, qh):
    mesh = plsc.VectorSubcoreMesh(core_axis_name="c", subcore_axis_name="s")

    @functools.partial(
        pl.kernel,
        out_type=(
            jax.ShapeDtypeStruct((NW, B), jnp.float32),
            jax.ShapeDtypeStruct((NW, B), jnp.float32),
            jax.ShapeDtypeStruct((NW, B, D), jnp.float32),
        ),
        mesh=mesh,
        scratch_types=[
            pltpu.VMEM((RPW,), jnp.int32),
            pltpu.VMEM((RPW,), jnp.float32),
            pltpu.VMEM((2, CH, D), jnp.float32),
            pltpu.SemaphoreType.DMA((2,)),
            pltpu.VMEM((B, D), jnp.float32),
            pltpu.VMEM((B,), jnp.float32),
            pltpu.VMEM((B,), jnp.float32),
            pltpu.SMEM((B,), jnp.float32),
            pltpu.SMEM((B,), jnp.float32),
        ],
        compiler_params=pltpu.CompilerParams(needs_layout_passes=False),
    )
    def k(x_hbm, ptr_hbm, qh_hbm, m_hbm, s_hbm, r_hbm,
          pv, ev, xa, sems, qr, mv, sv, msm, ssm):
        wid = lax.axis_index("s") * 2 + lax.axis_index("c")
        base = wid * RPW
        nch = lax.select(wid == NW - 1, RPW_LAST // CH, RPW // CH)
        lanes = lax.iota(jnp.int32, 16)

        pltpu.sync_copy(ptr_hbm.at[pl.ds(base, RPW)], pv)
        pltpu.sync_copy(qh_hbm, qr)

        def init_sm(i, _):
            msm[i] = jnp.float32(NEG)
            ssm[i] = jnp.float32(0.0)
            return 0
        lax.fori_loop(0, B, init_sm, 0)

        def stream_loop(process):
            pltpu.make_async_copy(
                x_hbm.at[pl.ds(base, CH)], xa.at[0], sems.at[0]).start()

            def body(ch, _):
                cur = lax.rem(ch, 2)
                nxt = 1 - cur

                @pl.when(ch + 1 < nch)
                def _():
                    pltpu.make_async_copy(
                        x_hbm.at[pl.ds(base + (ch + 1) * CH, CH)],
                        xa.at[nxt], sems.at[nxt]).start()
                pltpu.make_async_copy(
                    x_hbm.at[pl.ds(base + ch * CH, CH)],
                    xa.at[cur], sems.at[cur]).wait()
                process(cur, ch)
                return 0
            lax.fori_loop(0, nch, body, 0)

        def proc_a(slot, ch):
            for g2 in range(CH // 16):
                off = ch * CH + g2 * 16
                segv = pv[pl.ds(off, 16)]
                evec = jnp.zeros((16,), jnp.float32)
                for i in range(16):
                    seg = segv[i]
                    row = g2 * 16 + i
                    prods = []
                    for c in range(8):
                        xv = xa[slot, row, pl.ds(c * 16, 16)]
                        qv = qr[seg, pl.ds(c * 16, 16)]
                        prods.append(xv * qv)
                    s0 = (prods[0] + prods[1]) + (prods[2] + prods[3])
                    s1 = (prods[4] + prods[5]) + (prods[6] + prods[7])
                    tot = jnp.sum(s0 + s1)
                    msm[seg] = jnp.maximum(msm[seg], tot)
                    evec = jnp.where(lanes == i, tot, evec)
                ev[pl.ds(off, 16)] = evec
        stream_loop(proc_a)

        def pub_m(blk, _):
            vec = jnp.zeros((16,), jnp.float32)
            for j in range(16):
                vec = jnp.where(lanes == j, msm[blk * 16 + j], vec)
            mv[pl.ds(blk * 16, 16)] = vec
            return 0
        lax.fori_loop(0, B // 16, pub_m, 0)

        def pass_a2(g, _):
            off = g * 16
            segv = pv[pl.ds(off, 16)]
            evec = ev[pl.ds(off, 16)]
            mg = plsc.load_gather(mv, [segv])
            ex = jnp.exp(evec - mg)
            ev[pl.ds(off, 16)] = ex
            for i in range(16):
                seg = segv[i]
                ssm[seg] = ssm[seg] + ex[i]
            return 0
        lax.fori_loop(0, nch * (CH // 16), pass_a2, 0)

        def pub_s(blk, _):
            vec = jnp.zeros((16,), jnp.float32)
            for j in range(16):
                vec = jnp.where(lanes == j, ssm[blk * 16 + j], vec)
            sv[pl.ds(blk * 16, 16)] = vec
            return 0
        lax.fori_loop(0, B // 16, pub_s, 0)
        pltpu.sync_copy(mv, m_hbm.at[wid])
        pltpu.sync_copy(sv, s_hbm.at[wid])

        zero16 = jnp.zeros((16,), jnp.float32)

        def zero_r(row, _):
            for c in range(8):
                qr[row, pl.ds(c * 16, 16)] = zero16
            return 0
        lax.fori_loop(0, B, zero_r, 0)

        def proc_b(slot, ch):
            for g2 in range(CH // 16):
                off = ch * CH + g2 * 16
                segv = pv[pl.ds(off, 16)]
                exv = ev[pl.ds(off, 16)]
                for i in range(16):
                    seg = segv[i]
                    w = exv[i]
                    row = g2 * 16 + i
                    for c in range(8):
                        xv = xa[slot, row, pl.ds(c * 16, 16)]
                        qr[seg, pl.ds(c * 16, 16)] = (
                            qr[seg, pl.ds(c * 16, 16)] + w * xv)
        stream_loop(proc_b)
        pltpu.sync_copy(qr, r_hbm.at[wid])

    return k(x, ptr_pad, qh)


def _tc_step(m_all, s_all, r_all, qh_prev, h0, c0, h1, c1,
             W_ih0, W_hh0, b_ih0, b_hh0, W_ih1, W_hh1, b_ih1, b_hh1):
    def body(m_ref, s_ref, r_ref, qh_ref, h0_ref, c0_ref, h1_ref, c1_ref,
             wi0_ref, wh0_ref, bi0_ref, bh0_ref, wi1_ref, wh1_ref, bi1_ref,
             bh1_ref, q_ref, qh_o, h0_o, c0_o, h1_o, c1_o):
        m_all = m_ref[...]
        M = jnp.max(m_all, axis=0, keepdims=True)
        scale = jnp.exp(m_all - M)
        s_tot = jnp.sum(scale * s_ref[...], axis=0, keepdims=True)
        safe = s_tot > 0
        inv = jnp.where(safe, 1.0 / jnp.where(safe, s_tot, 1.0), 0.0)
        scale2 = scale * inv
        scale3 = lax.broadcast_in_dim(scale2, (NW, B, D), (0, 1))
        r = jnp.sum(scale3 * r_ref[...], axis=0)
        q = jnp.concatenate([qh_ref[...], r], axis=-1)
        q_ref[...] = q

        dn = (((1,), (1,)), ((), ()))
        g = (lax.dot_general(q, wi0_ref[...], dn,
                             preferred_element_type=jnp.float32)
             + lax.dot_general(h0_ref[...], wh0_ref[...], dn,
                               preferred_element_type=jnp.float32)
             + lax.broadcast_in_dim(bi0_ref[...] + bh0_ref[...],
                                    (B, 4 * D), (1,)))
        i0 = jax.nn.sigmoid(g[:, 0:D])
        f0 = jax.nn.sigmoid(g[:, D:2 * D])
        g0 = jnp.tanh(g[:, 2 * D:3 * D])
        o0 = jax.nn.sigmoid(g[:, 3 * D:4 * D])
        c0n = f0 * c0_ref[...] + i0 * g0
        h0n = o0 * jnp.tanh(c0n)
        g = (lax.dot_general(h0n, wi1_ref[...], dn,
                             preferred_element_type=jnp.float32)
             + lax.dot_general(h1_ref[...], wh1_ref[...], dn,
                               preferred_element_type=jnp.float32)
             + lax.broadcast_in_dim(bi1_ref[...] + bh1_ref[...],
                                    (B, 4 * D), (1,)))
        i1 = jax.nn.sigmoid(g[:, 0:D])
        f1 = jax.nn.sigmoid(g[:, D:2 * D])
        g1 = jnp.tanh(g[:, 2 * D:3 * D])
        o1 = jax.nn.sigmoid(g[:, 3 * D:4 * D])
        c1n = f1 * c1_ref[...] + i1 * g1
        h1n = o1 * jnp.tanh(c1n)
        qh_o[...] = h1n
        h0_o[...] = h0n
        c0_o[...] = c0n
        h1_o[...] = h1n
        c1_o[...] = c1n

    shapes = (
        jax.ShapeDtypeStruct((B, 2 * D), jnp.float32),
        jax.ShapeDtypeStruct((B, D), jnp.float32),
        jax.ShapeDtypeStruct((B, D), jnp.float32),
        jax.ShapeDtypeStruct((B, D), jnp.float32),
        jax.ShapeDtypeStruct((B, D), jnp.float32),
        jax.ShapeDtypeStruct((B, D), jnp.float32),
    )
    return pl.pallas_call(body, out_shape=shapes)(
        m_all, s_all, r_all, qh_prev, h0, c0, h1, c1,
        W_ih0, W_hh0, b_ih0, b_hh0, W_ih1, W_hh1, b_ih1, b_hh1)


def kernel(x, ptr, W_ih0, W_hh0, b_ih0, b_hh0, W_ih1, W_hh1, b_ih1, b_hh1):
    ptr32 = ptr.astype(jnp.int32)
    ptr_pad = jnp.concatenate(
        [ptr32, jnp.zeros((NW * RPW - N,), jnp.int32)])

    zM = jnp.full((NW, B), NEG, jnp.float32)
    zS = jnp.zeros((NW, B), jnp.float32)
    zR = jnp.zeros((NW, B, D), jnp.float32)
    qh = jnp.zeros((B, D), jnp.float32)
    h0 = c0 = h1 = c1 = jnp.zeros((B, D), jnp.float32)

    m_all, s_all, r_all = zM, zS, zR
    q = None
    for it in range(3):
        q, qh, h0, c0, h1, c1 = _tc_step(
            m_all, s_all, r_all, qh, h0, c0, h1, c1,
            W_ih0, W_hh0, b_ih0, b_hh0, W_ih1, W_hh1, b_ih1, b_hh1)
        m_all, s_all, r_all = _sc_readout(x, ptr_pad, qh)
    q, _, _, _, _, _ = _tc_step(
        m_all, s_all, r_all, qh, h0, c0, h1, c1,
        W_ih0, W_hh0, b_ih0, b_hh0, W_ih1, W_hh1, b_ih1, b_hh1)
    return q


# speedup vs baseline: 11.7597x; 2.0043x over previous
import functools
import jax
import jax.numpy as jnp
from jax import lax
from jax.experimental import pallas as pl
from jax.experimental.pallas import tpu as pltpu
from jax.experimental.pallas import tpu_sc as plsc

D = 128
B = 256
N = 100000
NW = 32
RPW = 3136
RPW_LAST = N - (NW - 1) * RPW
CH = 32
NEG = -3.0e38


def _sc_readout(x, ptr_pad, qh):
    mesh = plsc.VectorSubcoreMesh(core_axis_name="c", subcore_axis_name="s")

    @functools.partial(
        pl.kernel,
        out_type=(
            jax.ShapeDtypeStruct((NW, B), jnp.float32),
            jax.ShapeDtypeStruct((NW, B), jnp.float32),
            jax.ShapeDtypeStruct((NW, B, D), jnp.float32),
        ),
        mesh=mesh,
        scratch_types=[
            pltpu.VMEM((RPW,), jnp.int32),
            pltpu.VMEM((RPW,), jnp.float32),
            pltpu.VMEM((2, CH, D), jnp.float32),
            pltpu.SemaphoreType.DMA((2,)),
            pltpu.VMEM((B, D), jnp.float32),
            pltpu.VMEM((B,), jnp.float32),
            pltpu.VMEM((B,), jnp.float32),
            pltpu.SMEM((B,), jnp.float32),
            pltpu.SMEM((B,), jnp.float32),
        ],
        compiler_params=pltpu.CompilerParams(needs_layout_passes=False),
    )
    def k(x_hbm, ptr_hbm, qh_hbm, m_hbm, s_hbm, r_hbm,
          pv, ev, xa, sems, qr, mv, sv, msm, ssm):
        wid = lax.axis_index("s") * 2 + lax.axis_index("c")
        base = wid * RPW
        nch = lax.select(wid == NW - 1, RPW_LAST // CH, RPW // CH)
        lanes = lax.iota(jnp.int32, 16)

        pltpu.sync_copy(ptr_hbm.at[pl.ds(base, RPW)], pv)
        pltpu.sync_copy(qh_hbm, qr)

        def init_sm(i, _):
            msm[i] = jnp.float32(NEG)
            ssm[i] = jnp.float32(0.0)
            return 0
        lax.fori_loop(0, B, init_sm, 0)

        def stream_loop(process):
            pltpu.make_async_copy(
                x_hbm.at[pl.ds(base, CH)], xa.at[0], sems.at[0]).start()

            def body(ch, _):
                cur = lax.rem(ch, 2)
                nxt = 1 - cur

                @pl.when(ch + 1 < nch)
                def _():
                    pltpu.make_async_copy(
                        x_hbm.at[pl.ds(base + (ch + 1) * CH, CH)],
                        xa.at[nxt], sems.at[nxt]).start()
                pltpu.make_async_copy(
                    x_hbm.at[pl.ds(base + ch * CH, CH)],
                    xa.at[cur], sems.at[cur]).wait()
                process(cur, ch)
                return 0
            lax.fori_loop(0, nch, body, 0)

        def proc_a(slot, ch):
            for g2 in range(CH // 16):
                off = ch * CH + g2 * 16
                segv = pv[pl.ds(off, 16)]
                sa = segv[0]
                sb = segv[15]

                @pl.when(sa == sb)
                def _():
                    qc = [qr[sa, pl.ds(c * 16, 16)] for c in range(8)]
                    evec = jnp.zeros((16,), jnp.float32)
                    for i in range(16):
                        row = g2 * 16 + i
                        acc = xa[slot, row, pl.ds(0, 16)] * qc[0]
                        for c in range(1, 8):
                            acc = acc + xa[slot, row, pl.ds(c * 16, 16)] * qc[c]
                        evec = jnp.where(lanes == i, jnp.sum(acc), evec)
                    ev[pl.ds(off, 16)] = evec
                    msm[sa] = jnp.maximum(msm[sa], jnp.max(evec))

                @pl.when(sa != sb)
                def _():
                    evec = jnp.zeros((16,), jnp.float32)
                    for i in range(16):
                        seg = segv[i]
                        row = g2 * 16 + i
                        prods = []
                        for c in range(8):
                            xv = xa[slot, row, pl.ds(c * 16, 16)]
                            qv = qr[seg, pl.ds(c * 16, 16)]
                            prods.append(xv * qv)
                        s0 = (prods[0] + prods[1]) + (prods[2] + prods[3])
                        s1 = (prods[4] + prods[5]) + (prods[6] + prods[7])
                        tot = jnp.sum(s0 + s1)
                        msm[seg] = jnp.maximum(msm[seg], tot)
                        evec = jnp.where(lanes == i, tot, evec)
                    ev[pl.ds(off, 16)] = evec
        stream_loop(proc_a)

        def pub_m(blk, _):
            vec = jnp.zeros((16,), jnp.float32)
            for j in range(16):
                vec = jnp.where(lanes == j, msm[blk * 16 + j], vec)
            mv[pl.ds(blk * 16, 16)] = vec
            return 0
        lax.fori_loop(0, B // 16, pub_m, 0)

        def pass_a2(g, _):
            off = g * 16
            segv = pv[pl.ds(off, 16)]
            evec = ev[pl.ds(off, 16)]
            mg = plsc.load_gather(mv, [segv])
            ex = jnp.exp(evec - mg)
            ev[pl.ds(off, 16)] = ex
            sa = segv[0]
            sb = segv[15]

            @pl.when(sa == sb)
            def _():
                ssm[sa] = ssm[sa] + jnp.sum(ex)

            @pl.when(sa != sb)
            def _():
                for i in range(16):
                    seg = segv[i]
                    ssm[seg] = ssm[seg] + ex[i]
            return 0
        lax.fori_loop(0, nch * (CH // 16), pass_a2, 0)

        def pub_s(blk, _):
            vec = jnp.zeros((16,), jnp.float32)
            for j in range(16):
                vec = jnp.where(lanes == j, ssm[blk * 16 + j], vec)
            sv[pl.ds(blk * 16, 16)] = vec
            return 0
        lax.fori_loop(0, B // 16, pub_s, 0)
        pltpu.sync_copy(mv, m_hbm.at[wid])
        pltpu.sync_copy(sv, s_hbm.at[wid])

        zero16 = jnp.zeros((16,), jnp.float32)

        def zero_r(row, _):
            for c in range(8):
                qr[row, pl.ds(c * 16, 16)] = zero16
            return 0
        lax.fori_loop(0, B, zero_r, 0)

        def proc_b(slot, ch):
            for g2 in range(CH // 16):
                off = ch * CH + g2 * 16
                segv = pv[pl.ds(off, 16)]
                exv = ev[pl.ds(off, 16)]
                sa = segv[0]
                sb = segv[15]

                @pl.when(sa == sb)
                def _():
                    rc = [qr[sa, pl.ds(c * 16, 16)] for c in range(8)]
                    for i in range(16):
                        w = exv[i]
                        row = g2 * 16 + i
                        for c in range(8):
                            rc[c] = rc[c] + w * xa[slot, row,
                                                   pl.ds(c * 16, 16)]
                    for c in range(8):
                        qr[sa, pl.ds(c * 16, 16)] = rc[c]

                @pl.when(sa != sb)
                def _():
                    for i in range(16):
                        seg = segv[i]
                        w = exv[i]
                        row = g2 * 16 + i
                        for c in range(8):
                            xv = xa[slot, row, pl.ds(c * 16, 16)]
                            qr[seg, pl.ds(c * 16, 16)] = (
                                qr[seg, pl.ds(c * 16, 16)] + w * xv)
        stream_loop(proc_b)
        pltpu.sync_copy(qr, r_hbm.at[wid])

    return k(x, ptr_pad, qh)


def _tc_step(m_all, s_all, r_all, qh_prev, h0, c0, h1, c1,
             W_ih0, W_hh0, b_ih0, b_hh0, W_ih1, W_hh1, b_ih1, b_hh1):
    def body(m_ref, s_ref, r_ref, qh_ref, h0_ref, c0_ref, h1_ref, c1_ref,
             wi0_ref, wh0_ref, bi0_ref, bh0_ref, wi1_ref, wh1_ref, bi1_ref,
             bh1_ref, q_ref, qh_o, h0_o, c0_o, h1_o, c1_o):
        m_all = m_ref[...]
        M = jnp.max(m_all, axis=0, keepdims=True)
        scale = jnp.exp(m_all - M)
        s_tot = jnp.sum(scale * s_ref[...], axis=0, keepdims=True)
        safe = s_tot > 0
        inv = jnp.where(safe, 1.0 / jnp.where(safe, s_tot, 1.0), 0.0)
        scale2 = scale * inv
        scale3 = lax.broadcast_in_dim(scale2, (NW, B, D), (0, 1))
        r = jnp.sum(scale3 * r_ref[...], axis=0)
        q = jnp.concatenate([qh_ref[...], r], axis=-1)
        q_ref[...] = q

        dn = (((1,), (1,)), ((), ()))
        g = (lax.dot_general(q, wi0_ref[...], dn,
                             preferred_element_type=jnp.float32)
             + lax.dot_general(h0_ref[...], wh0_ref[...], dn,
                               preferred_element_type=jnp.float32)
             + lax.broadcast_in_dim(bi0_ref[...] + bh0_ref[...],
                                    (B, 4 * D), (1,)))
        i0 = jax.nn.sigmoid(g[:, 0:D])
        f0 = jax.nn.sigmoid(g[:, D:2 * D])
        g0 = jnp.tanh(g[:, 2 * D:3 * D])
        o0 = jax.nn.sigmoid(g[:, 3 * D:4 * D])
        c0n = f0 * c0_ref[...] + i0 * g0
        h0n = o0 * jnp.tanh(c0n)
        g = (lax.dot_general(h0n, wi1_ref[...], dn,
                             preferred_element_type=jnp.float32)
             + lax.dot_general(h1_ref[...], wh1_ref[...], dn,
                               preferred_element_type=jnp.float32)
             + lax.broadcast_in_dim(bi1_ref[...] + bh1_ref[...],
                                    (B, 4 * D), (1,)))
        i1 = jax.nn.sigmoid(g[:, 0:D])
        f1 = jax.nn.sigmoid(g[:, D:2 * D])
        g1 = jnp.tanh(g[:, 2 * D:3 * D])
        o1 = jax.nn.sigmoid(g[:, 3 * D:4 * D])
        c1n = f1 * c1_ref[...] + i1 * g1
        h1n = o1 * jnp.tanh(c1n)
        qh_o[...] = h1n
        h0_o[...] = h0n
        c0_o[...] = c0n
        h1_o[...] = h1n
        c1_o[...] = c1n

    shapes = (
        jax.ShapeDtypeStruct((B, 2 * D), jnp.float32),
        jax.ShapeDtypeStruct((B, D), jnp.float32),
        jax.ShapeDtypeStruct((B, D), jnp.float32),
        jax.ShapeDtypeStruct((B, D), jnp.float32),
        jax.ShapeDtypeStruct((B, D), jnp.float32),
        jax.ShapeDtypeStruct((B, D), jnp.float32),
    )
    return pl.pallas_call(body, out_shape=shapes)(
        m_all, s_all, r_all, qh_prev, h0, c0, h1, c1,
        W_ih0, W_hh0, b_ih0, b_hh0, W_ih1, W_hh1, b_ih1, b_hh1)


def kernel(x, ptr, W_ih0, W_hh0, b_ih0, b_hh0, W_ih1, W_hh1, b_ih1, b_hh1):
    ptr32 = ptr.astype(jnp.int32)
    ptr_pad = jnp.concatenate(
        [ptr32, jnp.zeros((NW * RPW - N,), jnp.int32)])

    zM = jnp.full((NW, B), NEG, jnp.float32)
    zS = jnp.zeros((NW, B), jnp.float32)
    zR = jnp.zeros((NW, B, D), jnp.float32)
    qh = jnp.zeros((B, D), jnp.float32)
    h0 = c0 = h1 = c1 = jnp.zeros((B, D), jnp.float32)

    m_all, s_all, r_all = zM, zS, zR
    q = None
    for it in range(3):
        q, qh, h0, c0, h1, c1 = _tc_step(
            m_all, s_all, r_all, qh, h0, c0, h1, c1,
            W_ih0, W_hh0, b_ih0, b_hh0, W_ih1, W_hh1, b_ih1, b_hh1)
        m_all, s_all, r_all = _sc_readout(x, ptr_pad, qh)
    q, _, _, _, _, _ = _tc_step(
        m_all, s_all, r_all, qh, h0, c0, h1, c1,
        W_ih0, W_hh0, b_ih0, b_hh0, W_ih1, W_hh1, b_ih1, b_hh1)
    return q


# speedup vs baseline: 17.9440x; 1.5259x over previous
import functools
import jax
import jax.numpy as jnp
from jax import lax
from jax.experimental import pallas as pl
from jax.experimental.pallas import tpu as pltpu
from jax.experimental.pallas import tpu_sc as plsc

D = 128
B = 256
N = 100000
NW = 32
RPW = 3136
RPW_LAST = N - (NW - 1) * RPW
CH = 32
NEG = -3.0e38


def _sc_readout(x, ptr_pad, qh):
    mesh = plsc.VectorSubcoreMesh(core_axis_name="c", subcore_axis_name="s")

    @functools.partial(
        pl.kernel,
        out_type=(
            jax.ShapeDtypeStruct((NW, B), jnp.float32),
            jax.ShapeDtypeStruct((NW, B), jnp.float32),
            jax.ShapeDtypeStruct((NW, B, D), jnp.float32),
        ),
        mesh=mesh,
        scratch_types=[
            pltpu.VMEM((RPW,), jnp.int32),
            pltpu.VMEM((2, CH, D), jnp.float32),
            pltpu.SemaphoreType.DMA((2,)),
            pltpu.VMEM((B, D), jnp.float32),
            pltpu.VMEM((B, D), jnp.float32),
            pltpu.VMEM((B,), jnp.float32),
            pltpu.VMEM((B,), jnp.float32),
            pltpu.SMEM((B,), jnp.float32),
            pltpu.SMEM((B,), jnp.float32),
        ],
        compiler_params=pltpu.CompilerParams(needs_layout_passes=False),
    )
    def k(x_hbm, ptr_hbm, qh_hbm, m_hbm, s_hbm, r_hbm,
          pv, xa, sems, qb, rb, mv, sv, msm, ssm):
        wid = lax.axis_index("s") * 2 + lax.axis_index("c")
        base = wid * RPW
        nch = lax.select(wid == NW - 1, RPW_LAST // CH, RPW // CH)
        lanes = lax.iota(jnp.int32, 16)
        masks = [lanes == i for i in range(16)]
        zero16 = jnp.zeros((16,), jnp.float32)

        pltpu.sync_copy(ptr_hbm.at[pl.ds(base, RPW)], pv)
        pltpu.sync_copy(qh_hbm, qb)

        def init_sm(i, _):
            msm[i] = jnp.float32(NEG)
            ssm[i] = jnp.float32(0.0)
            return 0
        lax.fori_loop(0, B, init_sm, 0)

        def zero_r(row, _):
            for c in range(8):
                rb[row, pl.ds(c * 16, 16)] = zero16
            return 0
        lax.fori_loop(0, B, zero_r, 0)

        def stream_loop(process):
            pltpu.make_async_copy(
                x_hbm.at[pl.ds(base, CH)], xa.at[0], sems.at[0]).start()

            def body(ch, _):
                cur = lax.rem(ch, 2)
                nxt = 1 - cur

                @pl.when(ch + 1 < nch)
                def _():
                    pltpu.make_async_copy(
                        x_hbm.at[pl.ds(base + (ch + 1) * CH, CH)],
                        xa.at[nxt], sems.at[nxt]).start()
                pltpu.make_async_copy(
                    x_hbm.at[pl.ds(base + ch * CH, CH)],
                    xa.at[cur], sems.at[cur]).wait()
                process(cur, ch)
                return 0
            lax.fori_loop(0, nch, body, 0)

        def proc(slot, ch):
            for g2 in range(CH // 16):
                off = ch * CH + g2 * 16
                segv = pv[pl.ds(off, 16)]
                sa = segv[0]
                sb = segv[15]

                @pl.when(sa == sb)
                def _():
                    qc = [qb[sa, pl.ds(c * 16, 16)] for c in range(8)]
                    tv = []
                    for i in range(16):
                        row = g2 * 16 + i
                        acc = xa[slot, row, pl.ds(0, 16)] * qc[0]
                        for c in range(1, 8):
                            acc = acc + xa[slot, row, pl.ds(c * 16, 16)] * qc[c]
                        tv.append(jnp.where(masks[i], jnp.sum(acc), 0.0))
                    while len(tv) > 1:
                        tv = [tv[j] + tv[j + 1] for j in range(0, len(tv), 2)]
                    evec = tv[0]
                    m_old = msm[sa]
                    mn = jnp.maximum(m_old, jnp.max(evec))
                    av = jnp.exp(jnp.full((16,), m_old - mn, jnp.float32))
                    exv = jnp.exp(evec - mn)
                    ssm[sa] = ssm[sa] * av[0] + jnp.sum(exv)
                    msm[sa] = mn
                    rc = [rb[sa, pl.ds(c * 16, 16)] * av for c in range(8)]
                    for i in range(16):
                        w = exv[i]
                        row = g2 * 16 + i
                        for c in range(8):
                            rc[c] = rc[c] + w * xa[slot, row,
                                                   pl.ds(c * 16, 16)]
                    for c in range(8):
                        rb[sa, pl.ds(c * 16, 16)] = rc[c]

                @pl.when(sa != sb)
                def _():
                    for i in range(16):
                        seg = segv[i]
                        row = g2 * 16 + i
                        xc = [xa[slot, row, pl.ds(c * 16, 16)]
                              for c in range(8)]
                        acc = xc[0] * qb[seg, pl.ds(0, 16)]
                        for c in range(1, 8):
                            acc = acc + xc[c] * qb[seg, pl.ds(c * 16, 16)]
                        e = jnp.sum(acc)
                        m_old = msm[seg]
                        mn = jnp.maximum(m_old, e)
                        av = jnp.exp(
                            jnp.full((16,), m_old - mn, jnp.float32))
                        wv = jnp.exp(jnp.full((16,), e - mn, jnp.float32))
                        ssm[seg] = ssm[seg] * av[0] + wv[0]
                        msm[seg] = mn
                        for c in range(8):
                            rb[seg, pl.ds(c * 16, 16)] = (
                                rb[seg, pl.ds(c * 16, 16)] * av + wv * xc[c])
        stream_loop(proc)

        def pub_ms(blk, _):
            vecm = jnp.zeros((16,), jnp.float32)
            vecs = jnp.zeros((16,), jnp.float32)
            for j in range(16):
                vecm = jnp.where(lanes == j, msm[blk * 16 + j], vecm)
                vecs = jnp.where(lanes == j, ssm[blk * 16 + j], vecs)
            mv[pl.ds(blk * 16, 16)] = vecm
            sv[pl.ds(blk * 16, 16)] = vecs
            return 0
        lax.fori_loop(0, B // 16, pub_ms, 0)
        pltpu.sync_copy(mv, m_hbm.at[wid])
        pltpu.sync_copy(sv, s_hbm.at[wid])
        pltpu.sync_copy(rb, r_hbm.at[wid])

    return k(x, ptr_pad, qh)


def _tc_step(m_all, s_all, r_all, qh_prev, h0, c0, h1, c1,
             W_ih0, W_hh0, b_ih0, b_hh0, W_ih1, W_hh1, b_ih1, b_hh1):
    def body(m_ref, s_ref, r_ref, qh_ref, h0_ref, c0_ref, h1_ref, c1_ref,
             wi0_ref, wh0_ref, bi0_ref, bh0_ref, wi1_ref, wh1_ref, bi1_ref,
             bh1_ref, q_ref, qh_o, h0_o, c0_o, h1_o, c1_o):
        m_all = m_ref[...]
        M = jnp.max(m_all, axis=0, keepdims=True)
        scale = jnp.exp(m_all - M)
        s_tot = jnp.sum(scale * s_ref[...], axis=0, keepdims=True)
        safe = s_tot > 0
        inv = jnp.where(safe, 1.0 / jnp.where(safe, s_tot, 1.0), 0.0)
        scale2 = scale * inv
        scale3 = lax.broadcast_in_dim(scale2, (NW, B, D), (0, 1))
        r = jnp.sum(scale3 * r_ref[...], axis=0)
        q = jnp.concatenate([qh_ref[...], r], axis=-1)
        q_ref[...] = q

        dn = (((1,), (1,)), ((), ()))
        g = (lax.dot_general(q, wi0_ref[...], dn,
                             preferred_element_type=jnp.float32)
             + lax.dot_general(h0_ref[...], wh0_ref[...], dn,
                               preferred_element_type=jnp.float32)
             + lax.broadcast_in_dim(bi0_ref[...] + bh0_ref[...],
                                    (B, 4 * D), (1,)))
        i0 = jax.nn.sigmoid(g[:, 0:D])
        f0 = jax.nn.sigmoid(g[:, D:2 * D])
        g0 = jnp.tanh(g[:, 2 * D:3 * D])
        o0 = jax.nn.sigmoid(g[:, 3 * D:4 * D])
        c0n = f0 * c0_ref[...] + i0 * g0
        h0n = o0 * jnp.tanh(c0n)
        g = (lax.dot_general(h0n, wi1_ref[...], dn,
                             preferred_element_type=jnp.float32)
             + lax.dot_general(h1_ref[...], wh1_ref[...], dn,
                               preferred_element_type=jnp.float32)
             + lax.broadcast_in_dim(bi1_ref[...] + bh1_ref[...],
                                    (B, 4 * D), (1,)))
        i1 = jax.nn.sigmoid(g[:, 0:D])
        f1 = jax.nn.sigmoid(g[:, D:2 * D])
        g1 = jnp.tanh(g[:, 2 * D:3 * D])
        o1 = jax.nn.sigmoid(g[:, 3 * D:4 * D])
        c1n = f1 * c1_ref[...] + i1 * g1
        h1n = o1 * jnp.tanh(c1n)
        qh_o[...] = h1n
        h0_o[...] = h0n
        c0_o[...] = c0n
        h1_o[...] = h1n
        c1_o[...] = c1n

    shapes = (
        jax.ShapeDtypeStruct((B, 2 * D), jnp.float32),
        jax.ShapeDtypeStruct((B, D), jnp.float32),
        jax.ShapeDtypeStruct((B, D), jnp.float32),
        jax.ShapeDtypeStruct((B, D), jnp.float32),
        jax.ShapeDtypeStruct((B, D), jnp.float32),
        jax.ShapeDtypeStruct((B, D), jnp.float32),
    )
    return pl.pallas_call(body, out_shape=shapes)(
        m_all, s_all, r_all, qh_prev, h0, c0, h1, c1,
        W_ih0, W_hh0, b_ih0, b_hh0, W_ih1, W_hh1, b_ih1, b_hh1)


def kernel(x, ptr, W_ih0, W_hh0, b_ih0, b_hh0, W_ih1, W_hh1, b_ih1, b_hh1):
    ptr32 = ptr.astype(jnp.int32)
    ptr_pad = jnp.concatenate(
        [ptr32, jnp.zeros((NW * RPW - N,), jnp.int32)])

    zM = jnp.full((NW, B), NEG, jnp.float32)
    zS = jnp.zeros((NW, B), jnp.float32)
    zR = jnp.zeros((NW, B, D), jnp.float32)
    qh = jnp.zeros((B, D), jnp.float32)
    h0 = c0 = h1 = c1 = jnp.zeros((B, D), jnp.float32)

    m_all, s_all, r_all = zM, zS, zR
    q = None
    for it in range(3):
        q, qh, h0, c0, h1, c1 = _tc_step(
            m_all, s_all, r_all, qh, h0, c0, h1, c1,
            W_ih0, W_hh0, b_ih0, b_hh0, W_ih1, W_hh1, b_ih1, b_hh1)
        m_all, s_all, r_all = _sc_readout(x, ptr_pad, qh)
    q, _, _, _, _, _ = _tc_step(
        m_all, s_all, r_all, qh, h0, c0, h1, c1,
        W_ih0, W_hh0, b_ih0, b_hh0, W_ih1, W_hh1, b_ih1, b_hh1)
    return q
